# baseline (speedup 1.0000x reference)
import functools

import jax
import jax.numpy as jnp
from jax import lax
from jax.experimental import pallas as pl
from jax.experimental.pallas import tpu as pltpu
from jax.experimental.pallas import tpu_sc as plsc

N = 10000
E = 320000
D = 128
L = 16
NC = 2
NS = 16
NW = NC * NS
EPT = E // NW
K = 80
NCHUNK = EPT // K
NPAD = 10240
RPT = NPAD // NS


def _scores_body(x_ref, wa_ref, wb_ref, si_ref, sj_ref):
    xv = x_ref[...]
    si_ref[...] = jnp.sum(xv * wa_ref[...], axis=1, keepdims=True)
    sj_ref[...] = jnp.sum(xv * wb_ref[...], axis=1, keepdims=True)


def _combine_body(p0_ref, p1_ref, d0_ref, d1_ref, o_ref):
    p = p0_ref[0] + p1_ref[0]
    d = d0_ref[0] + d1_ref[0] + 1e-16
    o_ref[...] = jnp.maximum(p / d, 0.0)


def _gat_sc(x_hbm, h_hbm, t_hbm, si_hbm, sj_hbm, outp_hbm, den_hbm,
            si_v, sj_v, rows0, rows1, ex0, ex1,
            h0, h1, t0, t1, out_sh, den_sh,
            sem_g0, sem_g1, sem_s0, sem_s1):
    cid = lax.axis_index("c")
    sid = lax.axis_index("s")
    wid = cid * NS + sid
    ebase = wid * EPT

    pltpu.sync_copy(si_hbm, si_v)
    pltpu.sync_copy(sj_hbm, sj_v)

    zeros16 = jnp.zeros((L,), jnp.float32)

    def _zrow(r, c_):
        for c in range(D // L):
            rows0[r, pl.ds(c * L, L)] = zeros16
        return c_

    lax.fori_loop(0, K, _zrow, 0)
    for i in range(K // L):
        ex0[pl.ds(i * L, L)] = zeros16

    rbase = sid * RPT
    for k in range(RPT // K):
        pltpu.sync_copy(rows0, out_sh.at[pl.ds(rbase + k * K, K)])
        pltpu.sync_copy(ex0, den_sh.at[pl.ds(rbase + k * K, K)])
    plsc.subcore_barrier()

    bufs = ((rows0, ex0, h0, t0, sem_g0, sem_s0),
            (rows1, ex1, h1, t1, sem_g1, sem_s1))

    def _start(j, b):
        rows_v, ex_v, h_v, t_v, sem_g, _ = bufs[b]
        base = ebase + j * K
        pltpu.sync_copy(h_hbm.at[pl.ds(base, K)], h_v)
        pltpu.sync_copy(t_hbm.at[pl.ds(base, K)], t_v)
        pltpu.async_copy(x_hbm.at[t_v], rows_v, sem_g)

    def _wait_scatter(b):
        rows_v, ex_v, h_v, _, _, sem_s = bufs[b]
        pltpu.make_async_copy(ex_v, den_sh.at[h_v], sem_s).wait()
        pltpu.make_async_copy(rows_v, out_sh.at[h_v], sem_s).wait()

    def _step(j, b, wait_pred, prefetch):
        bn = 1 - b
        rows_v, ex_v, h_v, t_v, sem_g, sem_s = bufs[b]
        for i in range(K // L):
            hv = h_v[pl.ds(i * L, L)]
            tv = t_v[pl.ds(i * L, L)]
            e = plsc.load_gather(si_v, [hv]) + plsc.load_gather(sj_v, [tv])
            le = jnp.where(e > 0.0, e, e * 0.01)
            ex_v[pl.ds(i * L, L)] = jnp.exp(le)
        pltpu.make_async_copy(x_hbm.at[t_v], rows_v, sem_g).wait()

        def _scale(i, cc_):
            exv = ex_v[pl.ds(i * L, L)]
            for jj in range(L):
                s = exv[jj]
                r = i * L + jj
                for c in range(D // L):
                    sl = pl.ds(c * L, L)
                    rows_v[r, sl] = rows_v[r, sl] * s
            return cc_

        lax.fori_loop(0, K // L, _scale, 0)
        if wait_pred is True:
            _wait_scatter(bn)
        elif wait_pred is not False:
            @pl.when(wait_pred)
            def _():
                _wait_scatter(bn)
        if prefetch:
            _start(j + 1, bn)
        pltpu.make_async_copy(ex_v, den_sh.at[h_v], sem_s).start(add=True)
        pltpu.make_async_copy(rows_v, out_sh.at[h_v], sem_s).start(add=True)

    _start(0, 0)

    def _pair(j2, c_):
        base = 2 * j2
        _step(base, 0, j2 > 0, True)
        _step(base + 1, 1, True, True)
        return c_

    lax.fori_loop(0, NCHUNK // 2, _pair, 0)
    _step(NCHUNK - 1, 0, True, False)
    _wait_scatter(0)
    plsc.subcore_barrier()

    pltpu.sync_copy(out_sh.at[pl.ds(rbase, RPT)],
                    outp_hbm.at[cid, pl.ds(rbase, RPT)])
    pltpu.sync_copy(den_sh.at[pl.ds(rbase, RPT)],
                    den_hbm.at[cid, pl.ds(rbase, RPT)])


_sc_call = functools.partial(
    pl.kernel,
    out_type=(jax.ShapeDtypeStruct((NC, NPAD, D), jnp.float32),
              jax.ShapeDtypeStruct((NC, NPAD), jnp.float32)),
    mesh=plsc.VectorSubcoreMesh(core_axis_name="c", subcore_axis_name="s"),
    compiler_params=pltpu.CompilerParams(needs_layout_passes=False),
    scratch_types=[
        pltpu.VMEM((N,), jnp.float32),
        pltpu.VMEM((N,), jnp.float32),
        pltpu.VMEM((K, D), jnp.float32),
        pltpu.VMEM((K, D), jnp.float32),
        pltpu.VMEM((K,), jnp.float32),
        pltpu.VMEM((K,), jnp.float32),
        pltpu.VMEM((K,), jnp.int32),
        pltpu.VMEM((K,), jnp.int32),
        pltpu.VMEM((K,), jnp.int32),
        pltpu.VMEM((K,), jnp.int32),
        pltpu.VMEM_SHARED((NPAD, D), jnp.float32),
        pltpu.VMEM_SHARED((NPAD,), jnp.float32),
        pltpu.SemaphoreType.DMA,
        pltpu.SemaphoreType.DMA,
        pltpu.SemaphoreType.DMA,
        pltpu.SemaphoreType.DMA,
    ],
)


def kernel(x, h, t, W_ai, W_aj):
    si, sj = pl.pallas_call(
        _scores_body,
        out_shape=(jax.ShapeDtypeStruct((N, 1), jnp.float32),
                   jax.ShapeDtypeStruct((N, 1), jnp.float32)),
    )(x, W_ai.reshape(1, D), W_aj.reshape(1, D))
    si = si.reshape(N)
    sj = sj.reshape(N)

    outp, den = _sc_call(_gat_sc)(x, h, t, si, sj)

    den3 = den.reshape(NC, NPAD, 1)
    out = pl.pallas_call(
        _combine_body,
        grid=(1,),
        in_specs=[
            pl.BlockSpec((1, N, D), lambda i: (0, 0, 0)),
            pl.BlockSpec((1, N, D), lambda i: (1, 0, 0)),
            pl.BlockSpec((1, N, 1), lambda i: (0, 0, 0)),
            pl.BlockSpec((1, N, 1), lambda i: (1, 0, 0)),
        ],
        out_specs=pl.BlockSpec((N, D), lambda i: (0, 0)),
        out_shape=jax.ShapeDtypeStruct((N, D), jnp.float32),
    )(outp, outp, den3, den3)
    return out


# speedup vs baseline: 1.0139x; 1.0139x over previous
import functools

import jax
import jax.numpy as jnp
from jax import lax
from jax.experimental import pallas as pl
from jax.experimental.pallas import tpu as pltpu
from jax.experimental.pallas import tpu_sc as plsc

N = 10000
E = 320000
D = 128
L = 16
NC = 2
NS = 16
NW = NC * NS
EPT = E // NW
K = 80
NCHUNK = EPT // K
NPAD = 10240
RPT = NPAD // NS


def _scores_body(x_ref, wa_ref, wb_ref, si_ref, sj_ref):
    xv = x_ref[...]
    si_ref[...] = jnp.sum(xv * wa_ref[...], axis=1, keepdims=True)
    sj_ref[...] = jnp.sum(xv * wb_ref[...], axis=1, keepdims=True)


def _combine_body(p0_ref, p1_ref, d0_ref, d1_ref, o_ref):
    p = p0_ref[0] + p1_ref[0]
    d = d0_ref[0] + d1_ref[0] + 1e-16
    o_ref[...] = jnp.maximum(p / d, 0.0)


def _gat_sc(x_hbm, h_hbm, t_hbm, tab_hbm, outp_hbm, den_hbm,
            tab_v, rows0, rows1, rows2, ex0, ex1, ex2,
            h0, h1, h2, t0, t1, t2, out_sh, den_sh,
            sem_g0, sem_g1, sem_g2, sem_s0, sem_s1, sem_s2):
    cid = lax.axis_index("c")
    sid = lax.axis_index("s")
    wid = cid * NS + sid
    ebase = wid * EPT

    pltpu.sync_copy(tab_hbm, tab_v)

    zeros16 = jnp.zeros((L,), jnp.float32)

    def _zrow(r, c_):
        for c in range(D // L):
            rows0[r, pl.ds(c * L, L)] = zeros16
        return c_

    lax.fori_loop(0, K, _zrow, 0)
    for i in range(K // L):
        ex0[pl.ds(i * L, L)] = zeros16

    rbase = sid * RPT
    for k in range(RPT // K):
        pltpu.sync_copy(rows0, out_sh.at[pl.ds(rbase + k * K, K)])
        pltpu.sync_copy(ex0, den_sh.at[pl.ds(rbase + k * K, K)])
    plsc.subcore_barrier()

    bufs = ((rows0, ex0, h0, t0, sem_g0, sem_s0),
            (rows1, ex1, h1, t1, sem_g1, sem_s1),
            (rows2, ex2, h2, t2, sem_g2, sem_s2))

    def _start(j, b):
        rows_v, ex_v, h_v, t_v, sem_g, _ = bufs[b]
        base = ebase + j * K
        pltpu.sync_copy(h_hbm.at[pl.ds(base, K)], h_v)
        pltpu.sync_copy(t_hbm.at[pl.ds(base, K)], t_v)
        pltpu.async_copy(x_hbm.at[t_v], rows_v, sem_g)

    def _wait_scatter(b):
        rows_v, ex_v, h_v, _, _, sem_s = bufs[b]
        pltpu.make_async_copy(ex_v, den_sh.at[h_v], sem_s).wait()
        pltpu.make_async_copy(rows_v, out_sh.at[h_v], sem_s).wait()

    def _step(j, b, wait_pred, prefetch):
        bn = (b + 1) % 3
        rows_v, ex_v, h_v, t_v, sem_g, sem_s = bufs[b]
        for i in range(K // L):
            hv = h_v[pl.ds(i * L, L)]
            tv = t_v[pl.ds(i * L, L)]
            ph = plsc.load_gather(tab_v, [hv])
            pt = plsc.load_gather(tab_v, [tv])
            si = plsc.bitcast(ph << 16, jnp.float32)
            sj = plsc.bitcast(pt & jnp.int32(-65536), jnp.float32)
            e = si + sj
            le = jnp.where(e > 0.0, e, e * 0.01)
            ex_v[pl.ds(i * L, L)] = jnp.exp(le)
        pltpu.make_async_copy(x_hbm.at[t_v], rows_v, sem_g).wait()

        def _scale(i, cc_):
            exv = ex_v[pl.ds(i * L, L)]
            for jj in range(L):
                s = exv[jj]
                r = i * L + jj
                for c in range(D // L):
                    sl = pl.ds(c * L, L)
                    rows_v[r, sl] = rows_v[r, sl] * s
            return cc_

        lax.fori_loop(0, K // L, _scale, 0)
        if wait_pred is True:
            _wait_scatter(bn)
        elif wait_pred is not False:
            @pl.when(wait_pred)
            def _():
                _wait_scatter(bn)
        if prefetch:
            _start(j + 1, bn)
        pltpu.make_async_copy(ex_v, den_sh.at[h_v], sem_s).start(add=True)
        pltpu.make_async_copy(rows_v, out_sh.at[h_v], sem_s).start(add=True)

    _start(0, 0)

    def _trip(j3, c_):
        base = 3 * j3
        for k in range(3):
            _step(base + k, k, (j3 > 0) if k < 2 else True, True)
        return c_

    lax.fori_loop(0, (NCHUNK - 2) // 3, _trip, 0)
    _step(NCHUNK - 2, 0, True, True)
    _step(NCHUNK - 1, 1, True, False)
    _wait_scatter(0)
    _wait_scatter(1)
    plsc.subcore_barrier()

    pltpu.sync_copy(out_sh.at[pl.ds(rbase, RPT)],
                    outp_hbm.at[cid, pl.ds(rbase, RPT)])
    pltpu.sync_copy(den_sh.at[pl.ds(rbase, RPT)],
                    den_hbm.at[cid, pl.ds(rbase, RPT)])


_sc_call = functools.partial(
    pl.kernel,
    out_type=(jax.ShapeDtypeStruct((NC, NPAD, D), jnp.float32),
              jax.ShapeDtypeStruct((NC, NPAD), jnp.float32)),
    mesh=plsc.VectorSubcoreMesh(core_axis_name="c", subcore_axis_name="s"),
    compiler_params=pltpu.CompilerParams(needs_layout_passes=False),
    scratch_types=[
        pltpu.VMEM((N,), jnp.int32),
        pltpu.VMEM((K, D), jnp.float32),
        pltpu.VMEM((K, D), jnp.float32),
        pltpu.VMEM((K, D), jnp.float32),
        pltpu.VMEM((K,), jnp.float32),
        pltpu.VMEM((K,), jnp.float32),
        pltpu.VMEM((K,), jnp.float32),
        pltpu.VMEM((K,), jnp.int32),
        pltpu.VMEM((K,), jnp.int32),
        pltpu.VMEM((K,), jnp.int32),
        pltpu.VMEM((K,), jnp.int32),
        pltpu.VMEM((K,), jnp.int32),
        pltpu.VMEM((K,), jnp.int32),
        pltpu.VMEM_SHARED((NPAD, D), jnp.float32),
        pltpu.VMEM_SHARED((NPAD,), jnp.float32),
        pltpu.SemaphoreType.DMA,
        pltpu.SemaphoreType.DMA,
        pltpu.SemaphoreType.DMA,
        pltpu.SemaphoreType.DMA,
        pltpu.SemaphoreType.DMA,
        pltpu.SemaphoreType.DMA,
    ],
)


def kernel(x, h, t, W_ai, W_aj):
    si, sj = pl.pallas_call(
        _scores_body,
        out_shape=(jax.ShapeDtypeStruct((N, 1), jnp.float32),
                   jax.ShapeDtypeStruct((N, 1), jnp.float32)),
    )(x, W_ai.reshape(1, D), W_aj.reshape(1, D))
    si_u = jax.lax.bitcast_convert_type(si.reshape(N), jnp.uint32)
    sj_u = jax.lax.bitcast_convert_type(sj.reshape(N), jnp.uint32)
    si_b = (si_u + 0x7FFF + ((si_u >> 16) & 1)) >> 16
    sj_b = (sj_u + 0x7FFF + ((sj_u >> 16) & 1)) & jnp.uint32(0xFFFF0000)
    tab = jax.lax.bitcast_convert_type(si_b | sj_b, jnp.int32)

    outp, den = _sc_call(_gat_sc)(x, h, t, tab)

    den3 = den.reshape(NC, NPAD, 1)
    out = pl.pallas_call(
        _combine_body,
        grid=(1,),
        in_specs=[
            pl.BlockSpec((1, N, D), lambda i: (0, 0, 0)),
            pl.BlockSpec((1, N, D), lambda i: (1, 0, 0)),
            pl.BlockSpec((1, N, 1), lambda i: (0, 0, 0)),
            pl.BlockSpec((1, N, 1), lambda i: (1, 0, 0)),
        ],
        out_specs=pl.BlockSpec((N, D), lambda i: (0, 0)),
        out_shape=jax.ShapeDtypeStruct((N, D), jnp.float32),
    )(outp, outp, den3, den3)
    return out


# speedup vs baseline: 2.0914x; 2.0627x over previous
import functools

import jax
import jax.numpy as jnp
from jax import lax
from jax.experimental import pallas as pl
from jax.experimental.pallas import tpu as pltpu
from jax.experimental.pallas import tpu_sc as plsc

N = 10000
E = 320000
D = 128
L = 16
NC = 2
NS = 16
NW = NC * NS
EPT = E // NW
K = 80
NCHUNK = EPT // K
NPAD = 10240
RPT = NPAD // NS


def _scores_body(x_ref, wa_ref, wb_ref, si_ref, sj_ref):
    xv = x_ref[...]
    si_ref[...] = jnp.sum(xv * wa_ref[...], axis=1, keepdims=True)
    sj_ref[...] = jnp.sum(xv * wb_ref[...], axis=1, keepdims=True)


def _combine_body(p0_ref, p1_ref, d0_ref, d1_ref, o_ref):
    p = p0_ref[0] + p1_ref[0]
    d = d0_ref[0] + d1_ref[0] + 1e-16
    o_ref[...] = jnp.maximum(p / d, 0.0)


def _gat_sc(x_hbm, h_hbm, t_hbm, tab_hbm, outp_hbm, den_hbm,
            tab_v, rows0, rows1, rows2, ex0, ex1, ex2,
            h0, h1, h2, t0, t1, t2, hs0, hs1, hs2, out_sh, den_sh,
            sem_g0, sem_g1, sem_g2, sem_s0, sem_s1, sem_s2,
            sem_i0, sem_i1, sem_i2):
    cid = lax.axis_index("c")
    sid = lax.axis_index("s")
    wid = cid * NS + sid
    ebase = wid * EPT

    pltpu.sync_copy(tab_hbm, tab_v)

    zeros16 = jnp.zeros((L,), jnp.float32)

    def _zrow(r, c_):
        for c in range(D // L):
            rows0[r, pl.ds(c * L, L)] = zeros16
        return c_

    lax.fori_loop(0, K, _zrow, 0)
    for i in range(K // L):
        ex0[pl.ds(i * L, L)] = zeros16

    rbase = sid * RPT
    for k in range(RPT // K):
        pltpu.sync_copy(rows0, out_sh.at[pl.ds(rbase + k * K, K)])
        pltpu.sync_copy(ex0, den_sh.at[pl.ds(rbase + k * K, K)])
    plsc.subcore_barrier()

    bufs = ((rows0, ex0, h0, t0, hs0, sem_g0, sem_s0, sem_i0),
            (rows1, ex1, h1, t1, hs1, sem_g1, sem_s1, sem_i1),
            (rows2, ex2, h2, t2, hs2, sem_g2, sem_s2, sem_i2))

    def _issue_idx(j, b):
        _, _, h_v, t_v, _, _, _, sem_i = bufs[b]
        base = ebase + j * K
        pltpu.make_async_copy(h_hbm.at[pl.ds(base, K)], h_v, sem_i).start()
        pltpu.make_async_copy(t_hbm.at[pl.ds(base, K)], t_v, sem_i).start()

    def _wait_idx(j, b):
        _, _, h_v, t_v, _, _, _, sem_i = bufs[b]
        base = ebase + j * K
        pltpu.make_async_copy(h_hbm.at[pl.ds(base, K)], h_v, sem_i).wait()
        pltpu.make_async_copy(t_hbm.at[pl.ds(base, K)], t_v, sem_i).wait()

    def _wait_scatter(b):
        rows_v, ex_v, _, _, hs_v, _, sem_s, _ = bufs[b]
        pltpu.make_async_copy(ex_v, den_sh.at[hs_v], sem_s).wait()
        pltpu.make_async_copy(rows_v, out_sh.at[hs_v], sem_s).wait()

    def _step(j, b, wait_pred, has_next, has_next2):
        bn = (b + 1) % 3
        bp = (b + 2) % 3
        rows_v, ex_v, h_v, t_v, hs_v, sem_g, sem_s, _ = bufs[b]
        if wait_pred is True:
            _wait_scatter(bn)
        elif wait_pred is not False:
            @pl.when(wait_pred)
            def _():
                _wait_scatter(bn)
        if has_next:
            _wait_idx(j + 1, bn)
            pltpu.async_copy(x_hbm.at[bufs[bn][3]], bufs[bn][0], bufs[bn][5])
        if has_next2:
            _issue_idx(j + 2, bp)
        for i in range(K // L):
            sl = pl.ds(i * L, L)
            hv = h_v[sl]
            tv = t_v[sl]
            ph = plsc.load_gather(tab_v, [hv])
            pt = plsc.load_gather(tab_v, [tv])
            si = plsc.bitcast(ph << 16, jnp.float32)
            sj = plsc.bitcast(pt & jnp.int32(-65536), jnp.float32)
            e = si + sj
            le = jnp.where(e > 0.0, e, e * 0.01)
            ex_v[sl] = jnp.exp(le)
            hs_v[sl] = hv
        pltpu.make_async_copy(x_hbm.at[t_v], rows_v, sem_g).wait()

        def _scale(i, cc_):
            exv = ex_v[pl.ds(i * L, L)]
            for jj in range(L):
                s = exv[jj]
                r = i * L + jj
                for c in range(D // L):
                    sl = pl.ds(c * L, L)
                    rows_v[r, sl] = rows_v[r, sl] * s
            return cc_

        lax.fori_loop(0, K // L, _scale, 0)
        pltpu.make_async_copy(ex_v, den_sh.at[hs_v], sem_s).start(add=True)
        pltpu.make_async_copy(rows_v, out_sh.at[hs_v], sem_s).start(add=True)

    _issue_idx(0, 0)
    _issue_idx(1, 1)
    _wait_idx(0, 0)
    pltpu.async_copy(x_hbm.at[t0], rows0, sem_g0)

    def _trip(j3, c_):
        base = 3 * j3
        for k in range(3):
            _step(base + k, k, (j3 > 0) if k < 2 else True, True, True)
        return c_

    lax.fori_loop(0, (NCHUNK - 2) // 3, _trip, 0)
    _step(NCHUNK - 2, 0, True, True, False)
    _step(NCHUNK - 1, 1, True, False, False)
    _wait_scatter(0)
    _wait_scatter(1)
    plsc.subcore_barrier()

    pltpu.sync_copy(out_sh.at[pl.ds(rbase, RPT)],
                    outp_hbm.at[cid, pl.ds(rbase, RPT)])
    pltpu.sync_copy(den_sh.at[pl.ds(rbase, RPT)],
                    den_hbm.at[cid, pl.ds(rbase, RPT)])


_sc_call = functools.partial(
    pl.kernel,
    out_type=(jax.ShapeDtypeStruct((NC, NPAD, D), jnp.float32),
              jax.ShapeDtypeStruct((NC, NPAD), jnp.float32)),
    mesh=plsc.VectorSubcoreMesh(core_axis_name="c", subcore_axis_name="s"),
    compiler_params=pltpu.CompilerParams(needs_layout_passes=False),
    scratch_types=[
        pltpu.VMEM((N,), jnp.int32),
        pltpu.VMEM((K, D), jnp.float32),
        pltpu.VMEM((K, D), jnp.float32),
        pltpu.VMEM((K, D), jnp.float32),
        pltpu.VMEM((K,), jnp.float32),
        pltpu.VMEM((K,), jnp.float32),
        pltpu.VMEM((K,), jnp.float32),
        pltpu.VMEM((K,), jnp.int32),
        pltpu.VMEM((K,), jnp.int32),
        pltpu.VMEM((K,), jnp.int32),
        pltpu.VMEM((K,), jnp.int32),
        pltpu.VMEM((K,), jnp.int32),
        pltpu.VMEM((K,), jnp.int32),
        pltpu.VMEM((K,), jnp.int32),
        pltpu.VMEM((K,), jnp.int32),
        pltpu.VMEM((K,), jnp.int32),
        pltpu.VMEM_SHARED((NPAD, D), jnp.float32),
        pltpu.VMEM_SHARED((NPAD,), jnp.float32),
        pltpu.SemaphoreType.DMA,
        pltpu.SemaphoreType.DMA,
        pltpu.SemaphoreType.DMA,
        pltpu.SemaphoreType.DMA,
        pltpu.SemaphoreType.DMA,
        pltpu.SemaphoreType.DMA,
        pltpu.SemaphoreType.DMA,
        pltpu.SemaphoreType.DMA,
        pltpu.SemaphoreType.DMA,
    ],
)


def kernel(x, h, t, W_ai, W_aj):
    si, sj = pl.pallas_call(
        _scores_body,
        out_shape=(jax.ShapeDtypeStruct((N, 1), jnp.float32),
                   jax.ShapeDtypeStruct((N, 1), jnp.float32)),
    )(x, W_ai.reshape(1, D), W_aj.reshape(1, D))
    si_u = jax.lax.bitcast_convert_type(si.reshape(N), jnp.uint32)
    sj_u = jax.lax.bitcast_convert_type(sj.reshape(N), jnp.uint32)
    si_b = (si_u + 0x7FFF + ((si_u >> 16) & 1)) >> 16
    sj_b = (sj_u + 0x7FFF + ((sj_u >> 16) & 1)) & jnp.uint32(0xFFFF0000)
    tab = jax.lax.bitcast_convert_type(si_b | sj_b, jnp.int32)

    outp, den = _sc_call(_gat_sc)(x, h, t, tab)

    den3 = den.reshape(NC, NPAD, 1)
    out = pl.pallas_call(
        _combine_body,
        grid=(1,),
        in_specs=[
            pl.BlockSpec((1, N, D), lambda i: (0, 0, 0)),
            pl.BlockSpec((1, N, D), lambda i: (1, 0, 0)),
            pl.BlockSpec((1, N, 1), lambda i: (0, 0, 0)),
            pl.BlockSpec((1, N, 1), lambda i: (1, 0, 0)),
        ],
        out_specs=pl.BlockSpec((N, D), lambda i: (0, 0)),
        out_shape=jax.ShapeDtypeStruct((N, D), jnp.float32),
    )(outp, outp, den3, den3)
    return out


# speedup vs baseline: 2.4649x; 1.1786x over previous
import functools

import jax
import jax.numpy as jnp
from jax import lax
from jax.experimental import pallas as pl
from jax.experimental.pallas import tpu as pltpu
from jax.experimental.pallas import tpu_sc as plsc

N = 10000
E = 320000
D = 128
L = 16
NC = 2
NS = 16
NW = NC * NS
EPT = E // NW
K = 80
NCHUNK = EPT // K
NPAD = 10240
RPT = NPAD // NS


def _scores_body(x_ref, wa_ref, wb_ref, tab_ref):
    xv = x_ref[...]
    si = jnp.sum(xv * wa_ref[...], axis=1, keepdims=True)
    sj = jnp.sum(xv * wb_ref[...], axis=1, keepdims=True)
    si_u = jax.lax.bitcast_convert_type(si, jnp.uint32)
    sj_u = jax.lax.bitcast_convert_type(sj, jnp.uint32)
    si_b = (si_u + 0x7FFF + ((si_u >> 16) & 1)) >> 16
    sj_b = (sj_u + 0x7FFF + ((sj_u >> 16) & 1)) & jnp.uint32(0xFFFF0000)
    tab_ref[...] = jax.lax.bitcast_convert_type(si_b | sj_b, jnp.int32)


def _combine_body(p0_ref, p1_ref, d0_ref, d1_ref, o_ref):
    p = p0_ref[0] + p1_ref[0]
    d = d0_ref[0] + d1_ref[0] + 1e-16
    o_ref[...] = jnp.maximum(p / d, 0.0)


def _gat_sc(x_hbm, h_hbm, t_hbm, tab_hbm, outp_hbm, den_hbm,
            tab_v, rows0, rows1, rows2, ex0, ex1, ex2,
            h0, h1, h2, t0, t1, t2, hs0, hs1, hs2, out_sh, den_sh,
            sem_g0, sem_g1, sem_g2, sem_s0, sem_s1, sem_s2,
            sem_i0, sem_i1, sem_i2):
    cid = lax.axis_index("c")
    sid = lax.axis_index("s")
    wid = cid * NS + sid
    ebase = wid * EPT

    pltpu.sync_copy(tab_hbm, tab_v)

    zeros16 = jnp.zeros((L,), jnp.float32)

    def _zrow(r, c_):
        for c in range(D // L):
            rows0[r, pl.ds(c * L, L)] = zeros16
        return c_

    lax.fori_loop(0, K, _zrow, 0)
    for i in range(K // L):
        ex0[pl.ds(i * L, L)] = zeros16

    rbase = sid * RPT
    for k in range(RPT // K):
        pltpu.sync_copy(rows0, out_sh.at[pl.ds(rbase + k * K, K)])
        pltpu.sync_copy(ex0, den_sh.at[pl.ds(rbase + k * K, K)])
    plsc.subcore_barrier()

    bufs = ((rows0, ex0, h0, t0, hs0, sem_g0, sem_s0, sem_i0),
            (rows1, ex1, h1, t1, hs1, sem_g1, sem_s1, sem_i1),
            (rows2, ex2, h2, t2, hs2, sem_g2, sem_s2, sem_i2))

    def _issue_idx(j, b):
        _, _, h_v, t_v, _, _, _, sem_i = bufs[b]
        base = ebase + j * K
        pltpu.make_async_copy(h_hbm.at[pl.ds(base, K)], h_v, sem_i).start()
        pltpu.make_async_copy(t_hbm.at[pl.ds(base, K)], t_v, sem_i).start()

    def _wait_idx(j, b):
        _, _, h_v, t_v, _, _, _, sem_i = bufs[b]
        base = ebase + j * K
        pltpu.make_async_copy(h_hbm.at[pl.ds(base, K)], h_v, sem_i).wait()
        pltpu.make_async_copy(t_hbm.at[pl.ds(base, K)], t_v, sem_i).wait()

    def _wait_scatter(b):
        rows_v, ex_v, _, _, hs_v, _, sem_s, _ = bufs[b]
        pltpu.make_async_copy(ex_v, den_sh.at[hs_v], sem_s).wait()
        pltpu.make_async_copy(rows_v, out_sh.at[hs_v], sem_s).wait()

    def _step(j, b, wait_pred, has_next, has_next2):
        bn = (b + 1) % 3
        bp = (b + 2) % 3
        rows_v, ex_v, h_v, t_v, hs_v, sem_g, sem_s, _ = bufs[b]
        if wait_pred is True:
            _wait_scatter(bn)
        elif wait_pred is not False:
            @pl.when(wait_pred)
            def _():
                _wait_scatter(bn)
        if has_next:
            _wait_idx(j + 1, bn)
            pltpu.async_copy(x_hbm.at[bufs[bn][3]], bufs[bn][0], bufs[bn][5])
        if has_next2:
            _issue_idx(j + 2, bp)
        for i in range(K // L):
            sl = pl.ds(i * L, L)
            hv = h_v[sl]
            tv = t_v[sl]
            ph = plsc.load_gather(tab_v, [hv])
            pt = plsc.load_gather(tab_v, [tv])
            si = plsc.bitcast(ph << 16, jnp.float32)
            sj = plsc.bitcast(pt & jnp.int32(-65536), jnp.float32)
            e = si + sj
            le = jnp.where(e > 0.0, e, e * 0.01)
            ex_v[sl] = jnp.exp(le)
            hs_v[sl] = hv
        pltpu.make_async_copy(x_hbm.at[t_v], rows_v, sem_g).wait()

        @functools.partial(plsc.parallel_loop, 0, K // L)
        def _(i):
            exv = ex_v[pl.ds(i * L, L)]
            for jj in range(L):
                s = exv[jj]
                r = i * L + jj
                for c in range(D // L):
                    sl = pl.ds(c * L, L)
                    rows_v[r, sl] = rows_v[r, sl] * s
        pltpu.make_async_copy(ex_v, den_sh.at[hs_v], sem_s).start(add=True)
        pltpu.make_async_copy(rows_v, out_sh.at[hs_v], sem_s).start(add=True)

    _issue_idx(0, 0)
    _issue_idx(1, 1)
    _wait_idx(0, 0)
    pltpu.async_copy(x_hbm.at[t0], rows0, sem_g0)

    def _trip(j3, c_):
        base = 3 * j3
        for k in range(3):
            _step(base + k, k, (j3 > 0) if k < 2 else True, True, True)
        return c_

    lax.fori_loop(0, (NCHUNK - 2) // 3, _trip, 0)
    _step(NCHUNK - 2, 0, True, True, False)
    _step(NCHUNK - 1, 1, True, False, False)
    _wait_scatter(0)
    _wait_scatter(1)
    plsc.subcore_barrier()

    pltpu.sync_copy(out_sh.at[pl.ds(rbase, RPT)],
                    outp_hbm.at[cid, pl.ds(rbase, RPT)])
    pltpu.sync_copy(den_sh.at[pl.ds(rbase, RPT)],
                    den_hbm.at[cid, pl.ds(rbase, RPT)])


_sc_call = functools.partial(
    pl.kernel,
    out_type=(jax.ShapeDtypeStruct((NC, NPAD, D), jnp.float32),
              jax.ShapeDtypeStruct((NC, NPAD), jnp.float32)),
    mesh=plsc.VectorSubcoreMesh(core_axis_name="c", subcore_axis_name="s"),
    compiler_params=pltpu.CompilerParams(needs_layout_passes=False),
    scratch_types=[
        pltpu.VMEM((N,), jnp.int32),
        pltpu.VMEM((K, D), jnp.float32),
        pltpu.VMEM((K, D), jnp.float32),
        pltpu.VMEM((K, D), jnp.float32),
        pltpu.VMEM((K,), jnp.float32),
        pltpu.VMEM((K,), jnp.float32),
        pltpu.VMEM((K,), jnp.float32),
        pltpu.VMEM((K,), jnp.int32),
        pltpu.VMEM((K,), jnp.int32),
        pltpu.VMEM((K,), jnp.int32),
        pltpu.VMEM((K,), jnp.int32),
        pltpu.VMEM((K,), jnp.int32),
        pltpu.VMEM((K,), jnp.int32),
        pltpu.VMEM((K,), jnp.int32),
        pltpu.VMEM((K,), jnp.int32),
        pltpu.VMEM((K,), jnp.int32),
        pltpu.VMEM_SHARED((NPAD, D), jnp.float32),
        pltpu.VMEM_SHARED((NPAD,), jnp.float32),
        pltpu.SemaphoreType.DMA,
        pltpu.SemaphoreType.DMA,
        pltpu.SemaphoreType.DMA,
        pltpu.SemaphoreType.DMA,
        pltpu.SemaphoreType.DMA,
        pltpu.SemaphoreType.DMA,
        pltpu.SemaphoreType.DMA,
        pltpu.SemaphoreType.DMA,
        pltpu.SemaphoreType.DMA,
    ],
)


def kernel(x, h, t, W_ai, W_aj):
    tab = pl.pallas_call(
        _scores_body,
        out_shape=jax.ShapeDtypeStruct((N, 1), jnp.int32),
    )(x, W_ai.reshape(1, D), W_aj.reshape(1, D))

    outp, den = _sc_call(_gat_sc)(x, h, t, tab.reshape(N))

    den3 = den.reshape(NC, NPAD, 1)
    out = pl.pallas_call(
        _combine_body,
        grid=(1,),
        in_specs=[
            pl.BlockSpec((1, N, D), lambda i: (0, 0, 0)),
            pl.BlockSpec((1, N, D), lambda i: (1, 0, 0)),
            pl.BlockSpec((1, N, 1), lambda i: (0, 0, 0)),
            pl.BlockSpec((1, N, 1), lambda i: (1, 0, 0)),
        ],
        out_specs=pl.BlockSpec((N, D), lambda i: (0, 0)),
        out_shape=jax.ShapeDtypeStruct((N, D), jnp.float32),
    )(outp, outp, den3, den3)
    return out
